# Initial kernel scaffold; baseline (speedup 1.0000x reference)
import jax
import jax.numpy as jnp
from jax.experimental import pallas as pl


def kernel(queries, keys):
    raise NotImplementedError("write your pallas kernel here")


# speedup vs baseline: 2.8309x; 2.8309x over previous
import functools

import jax
import jax.numpy as jnp
from jax.experimental import pallas as pl
from jax.experimental.pallas import tpu as pltpu

_TILE = 2048
_NEIGH = 3
_BIG = 1e30
_MASKED = 3e38


def _knn_body(q_ref, kT_ref, ksq_ref, ov_ref, oi_ref,
              v0s, v1s, v2s, i0s, i1s, i2s):
    j = pl.program_id(0)
    nt = pl.num_programs(0)
    Q = q_ref.shape[0]
    T = kT_ref.shape[1]

    @pl.when(j == 0)
    def _init():
        big = jnp.full((Q, 1), _BIG, jnp.float32)
        v0s[...] = big
        v1s[...] = big
        v2s[...] = big
        zero = jnp.zeros((Q, 1), jnp.int32)
        i0s[...] = zero
        i1s[...] = zero
        i2s[...] = zero

    q = q_ref[...]
    qsq = jnp.sum(q * q, axis=1, keepdims=True)
    ksq = ksq_ref[0]
    qk = jax.lax.dot_general(
        q.astype(jnp.bfloat16), kT_ref[...], (((1,), (0,)), ((), ())),
        preferred_element_type=jnp.float32)
    d = (qsq + ksq) - 2.0 * qk
    col = jax.lax.broadcasted_iota(jnp.int32, (Q, T), 1) + j * T

    v0, v1, v2 = v0s[...], v1s[...], v2s[...]
    i0, i1, i2 = i0s[...], i1s[...], i2s[...]
    for _ in range(_NEIGH):
        m = jnp.min(d, axis=1, keepdims=True)
        mi = jnp.min(jnp.where(d == m, col, jnp.int32(2**31 - 1)),
                     axis=1, keepdims=True)
        d = jnp.where(col == mi, jnp.float32(_MASKED), d)
        lt0 = m < v0
        lt1 = m < v1
        lt2 = m < v2
        v2n = jnp.where(lt1, v1, jnp.where(lt2, m, v2))
        i2n = jnp.where(lt1, i1, jnp.where(lt2, mi, i2))
        v1n = jnp.where(lt0, v0, jnp.where(lt1, m, v1))
        i1n = jnp.where(lt0, i0, jnp.where(lt1, mi, i1))
        v0n = jnp.where(lt0, m, v0)
        i0n = jnp.where(lt0, mi, i0)
        v0, v1, v2 = v0n, v1n, v2n
        i0, i1, i2 = i0n, i1n, i2n
    v0s[...], v1s[...], v2s[...] = v0, v1, v2
    i0s[...], i1s[...], i2s[...] = i0, i1, i2

    @pl.when(j == nt - 1)
    def _fin():
        li = jax.lax.broadcasted_iota(jnp.int32, (Q, 8), 1)
        anom = jnp.sqrt(jnp.maximum(v0, 0.0))
        ov_ref[...] = jnp.where(
            li == 0, v0, jnp.where(li == 1, v1, jnp.where(
                li == 2, v2, jnp.where(li == 3, anom, 0.0))))
        oi_ref[...] = jnp.where(
            li == 0, i0, jnp.where(li == 1, i1, jnp.where(li == 2, i2, 0)))


@jax.jit
def kernel(queries, keys):
    Q, D = queries.shape
    K = keys.shape[0]
    nt = -(-K // _TILE)
    kpad = nt * _TILE

    ksq = jnp.sum(keys * keys, axis=1)
    ksq_p = jnp.concatenate(
        [ksq, jnp.full((kpad - K,), _BIG, jnp.float32)]).reshape(nt, 1, _TILE)
    kT = jnp.pad(keys.astype(jnp.bfloat16).T, ((0, 0), (0, kpad - K)))

    ov, oi = pl.pallas_call(
        _knn_body,
        grid=(nt,),
        in_specs=[
            pl.BlockSpec((Q, D), lambda j: (0, 0)),
            pl.BlockSpec((D, _TILE), lambda j: (0, j)),
            pl.BlockSpec((1, 1, _TILE), lambda j: (j, 0, 0)),
        ],
        out_specs=[
            pl.BlockSpec((Q, 8), lambda j: (0, 0)),
            pl.BlockSpec((Q, 8), lambda j: (0, 0)),
        ],
        out_shape=[
            jax.ShapeDtypeStruct((Q, 8), jnp.float32),
            jax.ShapeDtypeStruct((Q, 8), jnp.int32),
        ],
        scratch_shapes=[
            pltpu.VMEM((Q, 1), jnp.float32),
            pltpu.VMEM((Q, 1), jnp.float32),
            pltpu.VMEM((Q, 1), jnp.float32),
            pltpu.VMEM((Q, 1), jnp.int32),
            pltpu.VMEM((Q, 1), jnp.int32),
            pltpu.VMEM((Q, 1), jnp.int32),
        ],
    )(queries, kT, ksq_p)
    return ov[:, :_NEIGH], oi[:, :_NEIGH], ov[:, _NEIGH]


# speedup vs baseline: 3.5308x; 1.2472x over previous
import jax
import jax.numpy as jnp
from jax.experimental import pallas as pl
from jax.experimental.pallas import tpu as pltpu

_TILE = 1024
_NEIGH = 3
_BIG = 1e30
_MASKED = 3e38
_IMAX = 2**31 - 1


def _extract3(vals, gidx):
    out = []
    for _ in range(_NEIGH):
        m = jnp.min(vals, axis=1, keepdims=True)
        mi = jnp.min(jnp.where(vals == m, gidx, jnp.int32(_IMAX)),
                     axis=1, keepdims=True)
        vals = jnp.where(gidx == mi, jnp.float32(_MASKED), vals)
        out.append((m, mi))
    return out


def _knn_body(q_ref, kT_ref, ksq_ref, ov_ref, oi_ref,
              a0s, a1s, a2s, t0s, t1s, t2s):
    j = pl.program_id(0)
    nt = pl.num_programs(0)
    Q = q_ref.shape[0]
    T = kT_ref.shape[1]

    @pl.when(j == 0)
    def _init():
        big = jnp.full((Q, T), _BIG, jnp.float32)
        a0s[...] = big
        a1s[...] = big
        a2s[...] = big
        zero = jnp.zeros((Q, T), jnp.int32)
        t0s[...] = zero
        t1s[...] = zero
        t2s[...] = zero

    q = q_ref[...]
    qsq = jnp.sum(q * q, axis=1, keepdims=True)
    ksq = ksq_ref[0]
    qk = jax.lax.dot_general(
        q.astype(jnp.bfloat16), kT_ref[...], (((1,), (0,)), ((), ())),
        preferred_element_type=jnp.float32)
    x = (qsq + ksq) - 2.0 * qk

    a0, a1, a2 = a0s[...], a1s[...], a2s[...]
    t0, t1, t2 = t0s[...], t1s[...], t2s[...]
    lt0 = x < a0
    lt1 = x < a1
    lt2 = x < a2
    a2s[...] = jnp.where(lt1, a1, jnp.where(lt2, x, a2))
    t2s[...] = jnp.where(lt1, t1, jnp.where(lt2, j, t2))
    a1s[...] = jnp.where(lt0, a0, jnp.where(lt1, x, a1))
    t1s[...] = jnp.where(lt0, t0, jnp.where(lt1, j, t1))
    a0s[...] = jnp.where(lt0, x, a0)
    t0s[...] = jnp.where(lt0, j, t0)

    @pl.when(j == nt - 1)
    def _fin():
        lane = jax.lax.broadcasted_iota(jnp.int32, (Q, T), 1)
        cands = []
        for aps, tps in ((a0s, t0s), (a1s, t1s), (a2s, t2s)):
            g = tps[...] * T + lane
            cands.extend(_extract3(aps[...], g))
        big = jnp.full((Q, 1), _MASKED, jnp.float32)
        imax = jnp.full((Q, 1), _IMAX, jnp.int32)
        v0 = v1 = v2 = big
        g0 = g1 = g2 = imax
        for cv, cg in cands:
            lt0 = (cv < v0) | ((cv == v0) & (cg < g0))
            lt1 = (cv < v1) | ((cv == v1) & (cg < g1))
            lt2 = (cv < v2) | ((cv == v2) & (cg < g2))
            v2n = jnp.where(lt1, v1, jnp.where(lt2, cv, v2))
            g2n = jnp.where(lt1, g1, jnp.where(lt2, cg, g2))
            v1n = jnp.where(lt0, v0, jnp.where(lt1, cv, v1))
            g1n = jnp.where(lt0, g0, jnp.where(lt1, cg, g1))
            v0n = jnp.where(lt0, cv, v0)
            g0n = jnp.where(lt0, cg, g0)
            v0, v1, v2 = v0n, v1n, v2n
            g0, g1, g2 = g0n, g1n, g2n
        li = jax.lax.broadcasted_iota(jnp.int32, (Q, 8), 1)
        anom = jnp.sqrt(jnp.maximum(v0, 0.0))
        ov_ref[...] = jnp.where(
            li == 0, v0, jnp.where(li == 1, v1, jnp.where(
                li == 2, v2, jnp.where(li == 3, anom, 0.0))))
        oi_ref[...] = jnp.where(
            li == 0, g0, jnp.where(li == 1, g1, jnp.where(li == 2, g2, 0)))


@jax.jit
def kernel(queries, keys):
    Q, D = queries.shape
    K = keys.shape[0]
    nt = -(-K // _TILE)
    kpad = nt * _TILE

    ksq = jnp.sum(keys * keys, axis=1)
    ksq_p = jnp.concatenate(
        [ksq, jnp.full((kpad - K,), _BIG, jnp.float32)]).reshape(nt, 1, _TILE)
    kT = jnp.pad(keys.astype(jnp.bfloat16).T, ((0, 0), (0, kpad - K)))

    ov, oi = pl.pallas_call(
        _knn_body,
        grid=(nt,),
        in_specs=[
            pl.BlockSpec((Q, D), lambda j: (0, 0)),
            pl.BlockSpec((D, _TILE), lambda j: (0, j)),
            pl.BlockSpec((1, 1, _TILE), lambda j: (j, 0, 0)),
        ],
        out_specs=[
            pl.BlockSpec((Q, 8), lambda j: (0, 0)),
            pl.BlockSpec((Q, 8), lambda j: (0, 0)),
        ],
        out_shape=[
            jax.ShapeDtypeStruct((Q, 8), jnp.float32),
            jax.ShapeDtypeStruct((Q, 8), jnp.int32),
        ],
        scratch_shapes=[
            pltpu.VMEM((Q, _TILE), jnp.float32),
            pltpu.VMEM((Q, _TILE), jnp.float32),
            pltpu.VMEM((Q, _TILE), jnp.float32),
            pltpu.VMEM((Q, _TILE), jnp.int32),
            pltpu.VMEM((Q, _TILE), jnp.int32),
            pltpu.VMEM((Q, _TILE), jnp.int32),
        ],
    )(queries, kT, ksq_p)
    return ov[:, :_NEIGH], oi[:, :_NEIGH], ov[:, _NEIGH]
